# baseline (speedup 1.0000x reference)
import functools

import jax
import jax.numpy as jnp
from jax import lax
from jax.experimental import pallas as pl
from jax.experimental.pallas import tpu as pltpu
from jax.experimental.pallas import tpu_sc as plsc

N = 16384
W = 50
D = 32
V = 1000001
VB = 2048
G = -(-V // VB)
VPAD = G * VB
NC = 2
NS = 16
NW = NC * NS
UPT = N // NW
C = 8
CW = C * W
NBLK = UPT // 128
INV_W = 1.0 / W
MAGIC = 1311


def _tc_linearize_body(in_ref, out_ref):
    x = in_ref[...]
    out_ref[...] = x.reshape(D, VB // 4, 4).transpose(1, 2, 0).reshape(VB // 4, 128)


_tc_linearize = pl.pallas_call(
    _tc_linearize_body,
    grid=(G,),
    in_specs=[pl.BlockSpec((D, VB), lambda g: (0, g))],
    out_specs=pl.BlockSpec((VB // 4, 128), lambda g: (g, 0)),
    out_shape=jax.ShapeDtypeStruct((VPAD // 4, 128), jnp.float32),
)


def _body(s4_hbm, pos_hbm, neg_hbm, out4_hbm,
          svb0, svb1, pc0, pc1, nc0, nc1, rows0, rows1, fm0, fm1,
          gsem0, gsem1, fsem0, fsem1, ssem):
    wid = lax.axis_index("s") * NC + lax.axis_index("c")
    ub0 = wid * NBLK

    iota16 = lax.iota(jnp.int32, 16)
    a_lo = lax.shift_right_logical(iota16, 3)
    r_v = iota16 & 7

    def load_state(b, svb):
        pltpu.async_copy(s4_hbm.at[:, ub0 + b], svb, ssem)

    def wait_state(svb):
        pltpu.make_async_copy(s4_hbm.at[:, 0], svb, ssem).wait()

    def build(svb, pc, nc, uc0):
        for i in range(CW // 16):
            j = iota16 + (i * 16)
            uu = lax.shift_right_logical(j * MAGIC, 16)
            w = j - uu * W
            t0 = uu * 0
            ucv = uu + uc0
            pc[pl.ds(i * 16, 16)] = plsc.load_gather(svb, [w, t0, ucv]) + 1
            nc[pl.ds(i * 16, 16)] = plsc.load_gather(svb, [w, t0 + 1, ucv]) + 1

    def fire(pc, nc, rows, sem):
        pltpu.async_copy(pos_hbm.at[pc], rows.at[pl.ds(0, CW)], sem)
        pltpu.async_copy(neg_hbm.at[nc], rows.at[pl.ds(CW, CW)], sem)

    def drain_gathers(rows, sem):
        pltpu.make_async_copy(pos_hbm.at[pl.ds(0, 2 * CW)], rows, sem).wait()

    def reduce_scatter(rows, fm, uc0):
        for uu in range(C):
            def rbody(r, accs):
                base = uu * W + r * 2
                a0, b0, a1, b1, a2, b2, a3, b3 = accs
                return (
                    a0 + rows[base, pl.ds(0, 16)],
                    b0 + rows[base, pl.ds(16, 16)],
                    a1 + rows[base + 1, pl.ds(0, 16)],
                    b1 + rows[base + 1, pl.ds(16, 16)],
                    a2 + rows[CW + base, pl.ds(0, 16)],
                    b2 + rows[CW + base, pl.ds(16, 16)],
                    a3 + rows[CW + base + 1, pl.ds(0, 16)],
                    b3 + rows[CW + base + 1, pl.ds(16, 16)],
                )

            z = jnp.zeros((16,), jnp.float32)
            a0, b0, a1, b1, a2, b2, a3, b3 = lax.fori_loop(
                0, W // 2, rbody, (z, z, z, z, z, z, z, z)
            )
            lo = ((a0 + a1) + (a2 + a3)) * INV_W
            hi = ((b0 + b1) + (b2 + b3)) * INV_W
            c_spl = iota16 * 0 + (uc0 + uu)
            plsc.store_scatter(fm, [a_lo, r_v, c_spl], lo)
            plsc.store_scatter(fm, [a_lo + 2, r_v, c_spl], hi)

    def flush(fm, b, fsem):
        pltpu.async_copy(fm, out4_hbm.at[:, ub0 + b], fsem)

    def wait_flush(fm, fsem):
        pltpu.make_async_copy(fm, out4_hbm.at[:, 0], fsem).wait()

    load_state(0, svb0)
    wait_state(svb0)
    build(svb0, pc0, nc0, 0)
    fire(pc0, nc0, rows0, gsem0)
    load_state(1, svb1)

    svb = (svb0, svb1)
    fm = (fm0, fm1)
    fsem = (fsem0, fsem1)

    for b in range(NBLK):
        p = b & 1
        svb_q = svb[b & 1]
        fm_p = fm[p]

        def ibody(ii, carry):
            uc0_0 = ii * 16
            uc0_1 = ii * 16 + 8
            build(svb_q, pc1, nc1, uc0_1)
            drain_gathers(rows0, gsem0)
            fire(pc1, nc1, rows1, gsem1)
            reduce_scatter(rows0, fm_p, uc0_0)
            pl.when(ii < 7)(lambda: build(svb_q, pc0, nc0, uc0_0 + 16))
            drain_gathers(rows1, gsem1)
            pl.when(ii < 7)(lambda: fire(pc0, nc0, rows0, gsem0))
            reduce_scatter(rows1, fm_p, uc0_1)
            return carry

        if b >= 2:
            wait_flush(fm_p, fsem[p])
        lax.fori_loop(0, 8, ibody, 0)
        flush(fm_p, b, fsem[p])
        if b < NBLK - 1:
            wait_state(svb[(b + 1) & 1])
            if b < NBLK - 2:
                load_state(b + 2, svb[b & 1])
            build(svb[(b + 1) & 1], pc0, nc0, 0)
            fire(pc0, nc0, rows0, gsem0)

    wait_flush(fm0, fsem0)
    wait_flush(fm1, fsem1)


_user_model_sc = functools.partial(
    pl.kernel,
    out_type=jax.ShapeDtypeStruct((4, 128, 8, 128), jnp.float32),
    mesh=plsc.VectorSubcoreMesh(core_axis_name="c", subcore_axis_name="s"),
    scratch_types=[
        pltpu.VMEM((W, 2, 128), jnp.int32),
        pltpu.VMEM((W, 2, 128), jnp.int32),
        pltpu.VMEM((CW,), jnp.int32),
        pltpu.VMEM((CW,), jnp.int32),
        pltpu.VMEM((CW,), jnp.int32),
        pltpu.VMEM((CW,), jnp.int32),
        pltpu.VMEM((2 * CW, D), jnp.float32),
        pltpu.VMEM((2 * CW, D), jnp.float32),
        pltpu.VMEM((4, 8, 128), jnp.float32),
        pltpu.VMEM((4, 8, 128), jnp.float32),
        pltpu.SemaphoreType.DMA,
        pltpu.SemaphoreType.DMA,
        pltpu.SemaphoreType.DMA,
        pltpu.SemaphoreType.DMA,
        pltpu.SemaphoreType.DMA,
    ],
    compiler_params=pltpu.CompilerParams(
        use_tc_tiling_on_sc=False, needs_layout_passes=False
    ),
)(_body)


def kernel(state, item_pos_emb, item_neg_emb):
    s4 = state.transpose(2, 1, 0).reshape(W, 2, 128, 128).transpose(0, 2, 1, 3)
    tbp = _tc_linearize(item_pos_emb.T).reshape(VPAD, D)
    tbn = _tc_linearize(item_neg_emb.T).reshape(VPAD, D)
    out4 = _user_model_sc(s4, tbp, tbn)
    return out4.transpose(1, 3, 0, 2).reshape(N, D)


# speedup vs baseline: 4.9364x; 4.9364x over previous
import functools

import jax
import jax.numpy as jnp
from jax import lax
from jax.experimental import pallas as pl
from jax.experimental.pallas import tpu as pltpu
from jax.experimental.pallas import tpu_sc as plsc

N = 16384
W = 50
D = 32
V = 1000001
VR = 1000008
TROWS = VR * 4
NC = 2
NS = 16
NW = NC * NS
UPT = N // NW
C = 8
CW = C * W
NBLK = UPT // 128
INV_W = 1.0 / W
MAGIC = 1311


def _body(s4_hbm, pos_hbm, neg_hbm, out4_hbm,
          svb0, svb1, pc0, pc1, nc0, nc1, rows0, rows1, fm0, fm1,
          gsem0, gsem1, fsem0, fsem1, ssem):
    wid = lax.axis_index("s") * NC + lax.axis_index("c")
    ub0 = wid * NBLK

    iota16 = lax.iota(jnp.int32, 16)
    a_lo = lax.shift_right_logical(iota16, 3)
    r_v = iota16 & 7

    def load_state(b, svb):
        pltpu.async_copy(s4_hbm.at[:, ub0 + b], svb, ssem)

    def wait_state(svb):
        pltpu.make_async_copy(s4_hbm.at[:, 0], svb, ssem).wait()

    def build(svb, pc, nc, uc0):
        for i in range(CW // 16):
            j = iota16 + (i * 16)
            uu = lax.shift_right_logical(j * MAGIC, 16)
            w = j - uu * W
            t0 = uu * 0
            ucv = uu + uc0
            pc[pl.ds(i * 16, 16)] = plsc.load_gather(svb, [w, t0, ucv]) * 4 + 4
            nc[pl.ds(i * 16, 16)] = plsc.load_gather(svb, [w, t0 + 1, ucv]) * 4 + 4

    def fire(pc, nc, rows, sem):
        pltpu.async_copy(pos_hbm.at[pc], rows.at[pl.ds(0, CW)], sem)
        pltpu.async_copy(neg_hbm.at[nc], rows.at[pl.ds(CW, CW)], sem)

    def drain_gathers(rows, sem):
        pltpu.make_async_copy(pos_hbm.at[pl.ds(0, 2 * CW)], rows, sem).wait()

    def reduce_scatter(rows, fm, uc0):
        for uu in range(C):
            def rbody(r, accs):
                base = uu * W + r * 2
                a0, b0, a1, b1, a2, b2, a3, b3 = accs
                return (
                    a0 + rows[base, pl.ds(0, 16)],
                    b0 + rows[base, pl.ds(16, 16)],
                    a1 + rows[base + 1, pl.ds(0, 16)],
                    b1 + rows[base + 1, pl.ds(16, 16)],
                    a2 + rows[CW + base, pl.ds(0, 16)],
                    b2 + rows[CW + base, pl.ds(16, 16)],
                    a3 + rows[CW + base + 1, pl.ds(0, 16)],
                    b3 + rows[CW + base + 1, pl.ds(16, 16)],
                )

            z = jnp.zeros((16,), jnp.float32)
            a0, b0, a1, b1, a2, b2, a3, b3 = lax.fori_loop(
                0, W // 2, rbody, (z, z, z, z, z, z, z, z)
            )
            lo = ((a0 + a1) + (a2 + a3)) * INV_W
            hi = ((b0 + b1) + (b2 + b3)) * INV_W
            c_spl = iota16 * 0 + (uc0 + uu)
            plsc.store_scatter(fm, [a_lo, r_v, c_spl], lo)
            plsc.store_scatter(fm, [a_lo + 2, r_v, c_spl], hi)

    def flush(fm, b, fsem):
        pltpu.async_copy(fm, out4_hbm.at[:, ub0 + b], fsem)

    def wait_flush(fm, fsem):
        pltpu.make_async_copy(fm, out4_hbm.at[:, 0], fsem).wait()

    load_state(0, svb0)
    wait_state(svb0)
    build(svb0, pc0, nc0, 0)
    fire(pc0, nc0, rows0, gsem0)
    load_state(1, svb1)

    svb = (svb0, svb1)
    fm = (fm0, fm1)
    fsem = (fsem0, fsem1)

    for b in range(NBLK):
        p = b & 1
        svb_q = svb[b & 1]
        fm_p = fm[p]

        def ibody(ii, carry):
            uc0_0 = ii * 16
            uc0_1 = ii * 16 + 8
            build(svb_q, pc1, nc1, uc0_1)
            drain_gathers(rows0, gsem0)
            fire(pc1, nc1, rows1, gsem1)
            reduce_scatter(rows0, fm_p, uc0_0)
            pl.when(ii < 7)(lambda: build(svb_q, pc0, nc0, uc0_0 + 16))
            drain_gathers(rows1, gsem1)
            pl.when(ii < 7)(lambda: fire(pc0, nc0, rows0, gsem0))
            reduce_scatter(rows1, fm_p, uc0_1)
            return carry

        if b >= 2:
            wait_flush(fm_p, fsem[p])
        lax.fori_loop(0, 8, ibody, 0)
        flush(fm_p, b, fsem[p])
        if b < NBLK - 1:
            wait_state(svb[(b + 1) & 1])
            if b < NBLK - 2:
                load_state(b + 2, svb[b & 1])
            build(svb[(b + 1) & 1], pc0, nc0, 0)
            fire(pc0, nc0, rows0, gsem0)

    wait_flush(fm0, fsem0)
    wait_flush(fm1, fsem1)


_user_model_sc = functools.partial(
    pl.kernel,
    out_type=jax.ShapeDtypeStruct((4, 128, 8, 128), jnp.float32),
    mesh=plsc.VectorSubcoreMesh(core_axis_name="c", subcore_axis_name="s"),
    scratch_types=[
        pltpu.VMEM((W, 2, 128), jnp.int32),
        pltpu.VMEM((W, 2, 128), jnp.int32),
        pltpu.VMEM((CW,), jnp.int32),
        pltpu.VMEM((CW,), jnp.int32),
        pltpu.VMEM((CW,), jnp.int32),
        pltpu.VMEM((CW,), jnp.int32),
        pltpu.VMEM((2 * CW, D), jnp.float32),
        pltpu.VMEM((2 * CW, D), jnp.float32),
        pltpu.VMEM((4, 8, 128), jnp.float32),
        pltpu.VMEM((4, 8, 128), jnp.float32),
        pltpu.SemaphoreType.DMA,
        pltpu.SemaphoreType.DMA,
        pltpu.SemaphoreType.DMA,
        pltpu.SemaphoreType.DMA,
        pltpu.SemaphoreType.DMA,
    ],
    compiler_params=pltpu.CompilerParams(
        use_tc_tiling_on_sc=False, needs_layout_passes=False
    ),
)(_body)


def kernel(state, item_pos_emb, item_neg_emb):
    s4 = state.transpose(2, 1, 0).reshape(W, 2, 128, 128).transpose(0, 2, 1, 3)
    tbp = jnp.pad(item_pos_emb, ((0, VR - V), (0, 128 - D))).reshape(TROWS, D)
    tbn = jnp.pad(item_neg_emb, ((0, VR - V), (0, 128 - D))).reshape(TROWS, D)
    out4 = _user_model_sc(s4, tbp, tbn)
    return out4.transpose(1, 3, 0, 2).reshape(N, D)
